# baseline (speedup 1.0000x reference)
import functools

import jax
import jax.numpy as jnp
from jax import lax
from jax.experimental import pallas as pl
from jax.experimental.pallas import tpu as pltpu

_BF = jnp.bfloat16
_F32 = jnp.float32


def _cell(g, c, H):
    sg = jax.nn.sigmoid(g[:, 0:3 * H])
    gg = jnp.tanh(g[:, 3 * H:4 * H])
    c_n = sg[:, H:2 * H] * c + sg[:, 0:H] * gg
    h_n = sg[:, 2 * H:3 * H] * jnp.tanh(c_n)
    return h_n, c_n


def _fused_kernel(lens_ref, x0_ref, x1_ref, x2_ref,
                  w00, w01, w02, w03, w04, w06, w07, w08, w09, w10,
                  w11, w12, w13, w14, w15, w16,
                  out_ref, sc0, sc1, sc2, *, T, BH, Hs, Hm, C):
    x_refs = (x0_ref, x1_ref, x2_ref)
    scs = (sc0, sc1, sc2)
    m_id = pl.program_id(1)

    lens = lens_ref[...]
    masks = [(lens > float(t)).astype(_F32) for t in range(T)]
    nmasks = [1.0 - mk for mk in masks]

    def cc(w, n, H):
        if H == Hm:
            return w
        return jnp.concatenate([w[..., j * Hm:j * Hm + H] for j in range(n)],
                               axis=-1)

    def rowcat(w, H):
        return jnp.concatenate([w[0:H], w[Hm:Hm + H]], axis=0)

    def step(s, gx, whh, st, H, sc):
        hf, cf, hb, cb = st
        t, tb = s, T - 1 - s
        G = 4 * H
        hcat = jnp.concatenate([hf, hb], axis=-1).astype(_BF)
        g_rec = jnp.dot(hcat, whh, preferred_element_type=_F32)
        gf = gx[t * BH:(t + 1) * BH, 0:G] + g_rec[:, 0:G]
        gb = gx[tb * BH:(tb + 1) * BH, G:2 * G] + g_rec[:, G:2 * G]
        hf_n, cf_n = _cell(gf, cf, H)
        hb_n, cb_n = _cell(gb, cb, H)
        if sc is not None:
            sc[t * BH:(t + 1) * BH, 0:H] = masks[t] * hf_n
            sc[tb * BH:(tb + 1) * BH, H:2 * H] = masks[tb] * hb_n
        hf = masks[t] * hf_n + nmasks[t] * hf
        cf = masks[t] * cf_n + nmasks[t] * cf
        hb = masks[tb] * hb_n + nmasks[tb] * hb
        cb = masks[tb] * cb_n + nmasks[tb] * cb
        return hf, cf, hb, cb

    for mm in range(3):
        @pl.when(m_id == mm)
        def _(mm=mm):
            H = Hs[mm]
            wih1 = cc(w00[0, 0:H], 8, H).astype(_BF)
            b1 = cc(w01[0], 8, H)
            whh1 = cc(rowcat(w02[0], H), 8, H).astype(_BF)
            lng = cc(w03[0], 2, H)
            lnb = cc(w04[0], 2, H)
            wih2 = cc(rowcat(w06[0], H), 8, H).astype(_BF)
            b2 = cc(w07[0], 8, H)
            whh2 = cc(rowcat(w08[0], H), 8, H).astype(_BF)
            w1 = jnp.concatenate([w09[0, q * Hm:q * Hm + H] for q in range(4)],
                                 axis=0).astype(_BF)

            x = x_refs[mm][...]
            x = jnp.swapaxes(x, 0, 1).reshape(T * BH, H).astype(_BF)

            gx1 = jnp.dot(x, wih1, preferred_element_type=_F32) + b1
            sc = scs[mm]
            st = tuple(jnp.zeros((BH, H), _F32) for _ in range(4))
            for s in range(T):
                st = step(s, gx1, whh1, st, H, sc)
            h1f, h1b = st[0], st[2]

            h1 = sc[...]
            mean = jnp.mean(h1, axis=-1, keepdims=True)
            cen = h1 - mean
            var = jnp.mean(cen * cen, axis=-1, keepdims=True)
            normed = cen * lax.rsqrt(var + 1e-5) * lng + lnb

            gx2 = jnp.dot(normed.astype(_BF), wih2,
                          preferred_element_type=_F32) + b2
            st = tuple(jnp.zeros((BH, H), _F32) for _ in range(4))
            for s in range(T):
                st = step(s, gx2, whh2, st, H, None)
            h2f, h2b = st[0], st[2]

            feats = jnp.concatenate([h1f, h2f, h1b, h2b], axis=-1).astype(_BF)
            h = jnp.maximum(jnp.dot(feats, w1,
                                    preferred_element_type=_F32) + w10[0], 0.0)
            h = jnp.maximum(jnp.dot(h.astype(_BF), w11[0].astype(_BF),
                                    preferred_element_type=_F32) + w12[0], 0.0)
            h = jnp.maximum(jnp.dot(h.astype(_BF), w13[0].astype(_BF),
                                    preferred_element_type=_F32) + w14[0], 0.0)
            logits = (jnp.dot(h.astype(_BF), w15[0].astype(_BF),
                              preferred_element_type=_F32) + w16[0]) * (1. / 3.)
            if mm == 0:
                out_ref[...] = logits
            else:
                out_ref[...] += logits


def kernel(w00, w01, w02, w03, w04, w05, w06, w07, w08, w09, w10,
           w11, w12, w13, w14, w15, w16,
           embed, sentences, video, acoustic, lengths):
    Hm = w02.shape[1] // 2
    C = w15.shape[2]
    B, T = sentences.shape
    BH = B // 2
    Hs = (embed.shape[1], video.shape[2], acoustic.shape[2])

    emb = embed[sentences].astype(_BF)
    lens_col = lengths.astype(_F32).reshape(B, 1)

    weights = (w00, w01, w02, w03, w04, w06, w07, w08, w09, w10,
               w11, w12, w13, w14, w15, w16)

    kfn = functools.partial(_fused_kernel, T=T, BH=BH, Hs=Hs, Hm=Hm, C=C)

    in_specs = [pl.BlockSpec((BH, 1), lambda i, m: (i, 0))]
    in_specs += [pl.BlockSpec((BH, T, H), lambda i, m: (i, 0, 0)) for H in Hs]
    in_specs += [pl.BlockSpec((1,) + w.shape[1:], lambda i, m: (m, 0, 0))
                 for w in weights]

    return pl.pallas_call(
        kfn,
        out_shape=jax.ShapeDtypeStruct((B, C), _F32),
        grid=(2, 3),
        in_specs=in_specs,
        out_specs=pl.BlockSpec((BH, C), lambda i, m: (i, 0)),
        scratch_shapes=[pltpu.VMEM((T * BH, 2 * H), _F32) for H in Hs],
        compiler_params=pltpu.CompilerParams(
            dimension_semantics=("parallel", "arbitrary")),
    )(lens_col, emb, video, acoustic, *weights)


# speedup vs baseline: 1.3707x; 1.3707x over previous
import functools

import jax
import jax.numpy as jnp
from jax import lax
from jax.experimental import pallas as pl
from jax.experimental.pallas import tpu as pltpu

_BF = jnp.bfloat16
_F32 = jnp.float32


def _fused_kernel(lens_ref, x0_ref, x1_ref, x2_ref,
                  w00, w01, w02, w03, w04, w05, w06, w07, w08, w09, w10,
                  w11, w12, w13, w14, w15, w16,
                  out_ref, xsc, h1sc, *, T, BH, Hs, Hm, C):
    m_id = pl.program_id(1)
    G = 4 * Hm

    lens = lens_ref[...]
    masks = [(lens > float(t)).astype(_F32) for t in range(T)]
    nmasks = [1.0 - mk for mk in masks]

    for mm, x_ref in enumerate((x0_ref, x1_ref, x2_ref)):
        @pl.when(m_id == mm)
        def _(x_ref=x_ref, H=Hs[mm]):
            x = x_ref[...].astype(_BF)
            x = jnp.swapaxes(x, 0, 1).reshape(T * BH, H)
            xsc[:, 0:H] = x
            if H < Hm:
                xsc[:, H:Hm] = jnp.zeros((T * BH, Hm - H), _BF)

    def cell(g, c):
        sg = jax.nn.sigmoid(g[:, 0:3 * Hm])
        gg = jnp.tanh(g[:, 3 * Hm:4 * Hm])
        c_n = sg[:, Hm:2 * Hm] * c + sg[:, 0:Hm] * gg
        h_n = sg[:, 2 * Hm:3 * Hm] * jnp.tanh(c_n)
        return h_n, c_n

    def step(s, gx, whh, st, collect):
        hf, cf, hb, cb = st
        t, tb = s, T - 1 - s
        hcat = jnp.concatenate([hf, hb], axis=-1).astype(_BF)
        g_rec = jnp.dot(hcat, whh, preferred_element_type=_F32)
        gf = gx[t * BH:(t + 1) * BH, 0:G] + g_rec[:, 0:G]
        gb = gx[tb * BH:(tb + 1) * BH, G:2 * G] + g_rec[:, G:2 * G]
        hf_n, cf_n = cell(gf, cf)
        hb_n, cb_n = cell(gb, cb)
        if collect:
            h1sc[t * BH:(t + 1) * BH, 0:Hm] = masks[t] * hf_n
            h1sc[tb * BH:(tb + 1) * BH, Hm:2 * Hm] = masks[tb] * hb_n
        hf = masks[t] * hf_n + nmasks[t] * hf
        cf = masks[t] * cf_n + nmasks[t] * cf
        hb = masks[tb] * hb_n + nmasks[tb] * hb
        cb = masks[tb] * cb_n + nmasks[tb] * cb
        return hf, cf, hb, cb

    gx1 = jnp.dot(xsc[...], w00[0].astype(_BF),
                  preferred_element_type=_F32) + w01[0]
    whh1 = w02[0].astype(_BF)
    st = tuple(jnp.zeros((BH, Hm), _F32) for _ in range(4))
    for s in range(T):
        st = step(s, gx1, whh1, st, True)
    h1f, h1b = st[0], st[2]

    h1 = h1sc[...]
    lnms = w05[0]
    mean = jnp.sum(h1 * lnms, axis=-1, keepdims=True)
    cen = h1 - mean
    var = jnp.sum(cen * cen * lnms, axis=-1, keepdims=True)
    normed = cen * lax.rsqrt(var + 1e-5) * w03[0] + w04[0]

    gx2 = jnp.dot(normed.astype(_BF), w06[0].astype(_BF),
                  preferred_element_type=_F32) + w07[0]
    whh2 = w08[0].astype(_BF)
    st = tuple(jnp.zeros((BH, Hm), _F32) for _ in range(4))
    for s in range(T):
        st = step(s, gx2, whh2, st, False)
    h2f, h2b = st[0], st[2]

    feats = jnp.concatenate([h1f, h2f, h1b, h2b], axis=-1).astype(_BF)
    h = jnp.maximum(jnp.dot(feats, w09[0].astype(_BF),
                            preferred_element_type=_F32) + w10[0], 0.0)
    h = jnp.maximum(jnp.dot(h.astype(_BF), w11[0].astype(_BF),
                            preferred_element_type=_F32) + w12[0], 0.0)
    h = jnp.maximum(jnp.dot(h.astype(_BF), w13[0].astype(_BF),
                            preferred_element_type=_F32) + w14[0], 0.0)
    logits = (jnp.dot(h.astype(_BF), w15[0].astype(_BF),
                      preferred_element_type=_F32) + w16[0]) * (1. / 3.)

    @pl.when(m_id == 0)
    def _():
        out_ref[...] = logits

    @pl.when(m_id != 0)
    def _():
        out_ref[...] += logits


def kernel(w00, w01, w02, w03, w04, w05, w06, w07, w08, w09, w10,
           w11, w12, w13, w14, w15, w16,
           embed, sentences, video, acoustic, lengths):
    Hm = w02.shape[1] // 2
    C = w15.shape[2]
    B, T = sentences.shape
    BH = B // 2
    Hs = (embed.shape[1], video.shape[2], acoustic.shape[2])

    emb = embed[sentences].astype(_BF)
    lens_col = lengths.astype(_F32).reshape(B, 1)

    weights = (w00, w01, w02, w03, w04, w05, w06, w07, w08, w09, w10,
               w11, w12, w13, w14, w15, w16)

    kfn = functools.partial(_fused_kernel, T=T, BH=BH, Hs=Hs, Hm=Hm, C=C)

    in_specs = [pl.BlockSpec((BH, 1), lambda i, m: (i, 0))]
    in_specs += [pl.BlockSpec((BH, T, H), lambda i, m: (i, 0, 0)) for H in Hs]
    in_specs += [pl.BlockSpec((1,) + w.shape[1:], lambda i, m: (m, 0, 0))
                 for w in weights]

    return pl.pallas_call(
        kfn,
        out_shape=jax.ShapeDtypeStruct((B, C), _F32),
        grid=(2, 3),
        in_specs=in_specs,
        out_specs=pl.BlockSpec((BH, C), lambda i, m: (i, 0)),
        scratch_shapes=[pltpu.VMEM((T * BH, Hm), _BF),
                        pltpu.VMEM((T * BH, 2 * Hm), _F32)],
        compiler_params=pltpu.CompilerParams(
            dimension_semantics=("parallel", "arbitrary")),
    )(lens_col, emb, video, acoustic, *weights)
